# baseline (speedup 1.0000x reference)
import functools

import jax
import jax.numpy as jnp
from jax import lax
from jax.experimental import pallas as pl
from jax.experimental.pallas import tpu as pltpu
from jax.experimental.pallas import tpu_sc as plsc

_PROB = jnp.array([0.25, 0.25, 0.5], dtype=jnp.float32)


def _switch_affine_kernel(idx_ref, x_ref, o_ref):
    idx = idx_ref[0]
    a = jnp.where(idx == 0, 2.0, jnp.where(idx == 1, 1.0, -1.0)).astype(jnp.float32)
    b = jnp.where(idx == 1, 1.0, 0.0).astype(jnp.float32)
    o_ref[...] = x_ref[...] * a + b


def _tc_apply(idx, x2):
    rows, cols = x2.shape
    block_rows = 2048
    grid = rows // block_rows
    return pl.pallas_call(
        _switch_affine_kernel,
        grid_spec=pltpu.PrefetchScalarGridSpec(
            num_scalar_prefetch=1,
            grid=(grid,),
            in_specs=[pl.BlockSpec((block_rows, cols), lambda i, s: (i, 0))],
            out_specs=pl.BlockSpec((block_rows, cols), lambda i, s: (i, 0)),
        ),
        out_shape=jax.ShapeDtypeStruct(x2.shape, x2.dtype),
    )(idx.reshape(1), x2)


_NC = 2
_NS = 16
_NW = _NC * _NS
_LANES = 16
_CH = 65536


def _sc_body(x_hbm, a_hbm, b_hbm, out_hbm, buf, av, bv):
    total = x_hbm.shape[0]
    per_w = total // _NW
    wid = lax.axis_index("s") * _NC + lax.axis_index("c")
    base = wid * per_w
    pltpu.sync_copy(a_hbm, av)
    pltpu.sync_copy(b_hbm, bv)
    a = av[...]
    b = bv[...]
    unroll = 8
    span = unroll * _LANES

    def chunk_body(g, carry):
        off = base + g * _CH
        pltpu.sync_copy(x_hbm.at[pl.ds(off, _CH)], buf)

        def inner(i, c):
            start = i * span
            for u in range(unroll):
                s = pl.ds(start + u * _LANES, _LANES)
                buf[s] = buf[s] * a + b
            return c

        lax.fori_loop(0, _CH // span, inner, 0)
        pltpu.sync_copy(buf, out_hbm.at[pl.ds(off, _CH)])
        return carry

    lax.fori_loop(0, per_w // _CH, chunk_body, 0)


def _sc_apply(a16, b16, xf):
    mesh = plsc.VectorSubcoreMesh(core_axis_name="c", subcore_axis_name="s")
    fn = functools.partial(
        pl.kernel,
        mesh=mesh,
        out_type=jax.ShapeDtypeStruct(xf.shape, jnp.float32),
        scratch_types=[
            pltpu.VMEM((_CH,), jnp.float32),
            pltpu.VMEM((_LANES,), jnp.float32),
            pltpu.VMEM((_LANES,), jnp.float32),
        ],
    )(_sc_body)
    return fn(xf, a16, b16)


def kernel(x):
    idx = jax.random.categorical(jax.random.key(42), jnp.log(_PROB)).astype(jnp.int32)
    a = jnp.where(idx == 0, 2.0, jnp.where(idx == 1, 1.0, -1.0)).astype(jnp.float32)
    b = jnp.where(idx == 1, 1.0, 0.0).astype(jnp.float32)
    a16 = jnp.full((_LANES,), 1.0, jnp.float32) * a
    b16 = jnp.full((_LANES,), 1.0, jnp.float32) * b

    shape = x.shape
    xf = x.reshape(-1)
    out = _sc_apply(a16, b16, xf)
    return out.reshape(shape)


# speedup vs baseline: 1.3016x; 1.3016x over previous
import functools

import jax
import jax.numpy as jnp
from jax import lax
from jax.experimental import pallas as pl
from jax.experimental.pallas import tpu as pltpu
from jax.experimental.pallas import tpu_sc as plsc

_PROB = jnp.array([0.25, 0.25, 0.5], dtype=jnp.float32)


def _switch_affine_kernel(idx_ref, x_ref, o_ref):
    idx = idx_ref[0]
    a = jnp.where(idx == 0, 2.0, jnp.where(idx == 1, 1.0, -1.0)).astype(jnp.float32)
    b = jnp.where(idx == 1, 1.0, 0.0).astype(jnp.float32)
    o_ref[...] = x_ref[...] * a + b


def _tc_apply(idx, x2, out_rows):
    rows, cols = x2.shape
    block_rows = 2048
    grid = out_rows // block_rows
    return pl.pallas_call(
        _switch_affine_kernel,
        grid_spec=pltpu.PrefetchScalarGridSpec(
            num_scalar_prefetch=1,
            grid=(grid,),
            in_specs=[pl.BlockSpec((block_rows, cols), lambda i, s: (i, 0))],
            out_specs=pl.BlockSpec((block_rows, cols), lambda i, s: (i, 0)),
        ),
        out_shape=jax.ShapeDtypeStruct((out_rows, cols), x2.dtype),
    )(idx.reshape(1), x2)


_NC = 2
_NS = 16
_NW = _NC * _NS
_LANES = 16
_CH = 65536


def _sc_body(x_hbm, a_hbm, b_hbm, out_hbm, buf, av, bv, *, offset, size):
    per_w = size // _NW
    wid = lax.axis_index("s") * _NC + lax.axis_index("c")
    base = offset + wid * per_w
    out_base = wid * per_w
    pltpu.sync_copy(a_hbm, av)
    pltpu.sync_copy(b_hbm, bv)
    a = av[...]
    b = bv[...]
    unroll = 8
    span = unroll * _LANES

    def chunk_body(g, carry):
        off = base + g * _CH
        pltpu.sync_copy(x_hbm.at[pl.ds(off, _CH)], buf)

        def inner(i, c):
            start = i * span
            for u in range(unroll):
                s = pl.ds(start + u * _LANES, _LANES)
                buf[s] = buf[s] * a + b
            return c

        lax.fori_loop(0, _CH // span, inner, 0)
        pltpu.sync_copy(buf, out_hbm.at[pl.ds(out_base + g * _CH, _CH)])
        return carry

    lax.fori_loop(0, per_w // _CH, chunk_body, 0)


def _sc_apply(a16, b16, xf, offset, size):
    mesh = plsc.VectorSubcoreMesh(core_axis_name="c", subcore_axis_name="s")
    fn = functools.partial(
        pl.kernel,
        mesh=mesh,
        out_type=jax.ShapeDtypeStruct((size,), jnp.float32),
        scratch_types=[
            pltpu.VMEM((_CH,), jnp.float32),
            pltpu.VMEM((_LANES,), jnp.float32),
            pltpu.VMEM((_LANES,), jnp.float32),
        ],
    )(functools.partial(_sc_body, offset=offset, size=size))
    return fn(xf, a16, b16)


def kernel(x):
    idx = jax.random.categorical(jax.random.key(42), jnp.log(_PROB)).astype(jnp.int32)
    a = jnp.where(idx == 0, 2.0, jnp.where(idx == 1, 1.0, -1.0)).astype(jnp.float32)
    b = jnp.where(idx == 1, 1.0, 0.0).astype(jnp.float32)
    a16 = jnp.full((_LANES,), 1.0, jnp.float32) * a
    b16 = jnp.full((_LANES,), 1.0, jnp.float32) * b

    shape = x.shape
    cols = shape[-1]
    x2 = x.reshape(-1, cols)
    rows = x2.shape[0]

    sc_rows = 2048
    tc_rows = rows - sc_rows

    tc_out = _tc_apply(idx, x2, tc_rows)
    sc_out = _sc_apply(a16, b16, x2.reshape(-1), tc_rows * cols, sc_rows * cols)
    out = jnp.concatenate([tc_out, sc_out.reshape(sc_rows, cols)], axis=0)
    return out.reshape(shape)
